# baseline (speedup 1.0000x reference)
import functools

import numpy as np

import jax
import jax.numpy as jnp
from jax import lax
from jax.experimental import pallas as pl
from jax.experimental.pallas import tpu as pltpu
from jax.experimental.pallas import tpu_sc as plsc

N = 10000
E = 320000
D = 128

NC = 2
NS = 16
NW = NC * NS
LANES = 16

CHUNK = 80
CPW = E // (NW * CHUNK)
PH = 5
PC = CPW // PH
WB = 1000
WBT = N // WB
SPLA = 48
SPLB = CHUNK - SPLA
DW = D // 2


def _sc_aggregate(v_pk, col2d, rowa, rowb, a2d, zeros):
    mesh = plsc.VectorSubcoreMesh(
        core_axis_name="c", subcore_axis_name="s", num_cores=NC, num_subcores=NS
    )

    @functools.partial(
        pl.kernel,
        out_type=jax.ShapeDtypeStruct((NC, N, D), jnp.float32),
        mesh=mesh,
        scratch_types=[
            pltpu.VMEM((PC, CHUNK), jnp.int32),
            pltpu.VMEM((PC, SPLA), jnp.int32),
            pltpu.VMEM((PC, SPLB), jnp.int32),
            pltpu.VMEM((PC, CHUNK), jnp.float32),
            pltpu.VMEM((CHUNK, DW), jnp.int32),
            pltpu.VMEM((CHUNK, DW), jnp.int32),
            pltpu.VMEM((SPLA, D), jnp.float32),
            pltpu.VMEM((SPLB, D), jnp.float32),
            pltpu.VMEM_SHARED((N, D), jnp.float32),
            pltpu.SemaphoreType.DMA,
            pltpu.SemaphoreType.DMA,
            pltpu.SemaphoreType.DMA,
            pltpu.SemaphoreType.DMA,
        ],
        compiler_params=pltpu.CompilerParams(use_tc_tiling_on_sc=False),
    )
    def agg(v_hbm, col_hbm, rowa_hbm, rowb_hbm, a_hbm, zeros_hbm, out_hbm,
            colv, rowva, rowvb, av, bf0, bf1, fa, fb, acc,
            semg0, semg1, semsa, semsb):
        cid = lax.axis_index("c")
        sid = lax.axis_index("s")
        wid = cid * NS + sid

        @pl.when(sid < WBT)
        def _():
            pltpu.sync_copy(zeros_hbm, acc.at[pl.ds(sid * WB, WB)])

        plsc.subcore_barrier()

        MASK = jnp.int32(-65536)

        def expand_half(bb, j, dst, g_lo, g_hi, e_off):
            def group_body(g, c2):
                a_vec = av[j, pl.ds(g * LANES, LANES)]
                for i in range(LANES):
                    s = jnp.full((LANES,), a_vec[i], jnp.float32)
                    e = g * LANES + i
                    eo = e - e_off
                    for t in range(DW // LANES):
                        u = bb[e, pl.ds(t * LANES, LANES)]
                        lo = lax.bitcast_convert_type(lax.shift_left(u, 16), jnp.float32)
                        hi = lax.bitcast_convert_type(jnp.bitwise_and(u, MASK), jnp.float32)
                        dst[eo, pl.ds(t * LANES, LANES)] = lo * s
                        dst[eo, pl.ds(DW + t * LANES, LANES)] = hi * s
                return c2

            lax.fori_loop(g_lo, g_hi, group_body, 0)

        def gather(j, bb, sem):
            pltpu.async_copy(v_hbm.at[colv.at[j]], bb, sem)

        def gather_wait(j, bb, sem):
            pltpu.make_async_copy(v_hbm.at[colv.at[j]], bb, sem).wait()

        def process(bb, j):
            @pl.when(j > 0)
            def _():
                pltpu.make_async_copy(fa, acc.at[rowva.at[j]], semsa).wait()

            expand_half(bb, j, fa, 0, SPLA // LANES, 0)
            pltpu.async_copy(fa, acc.at[rowva.at[j]], semsa, add=True)

            @pl.when(j > 0)
            def _():
                pltpu.make_async_copy(fb, acc.at[rowvb.at[j]], semsb).wait()

            expand_half(bb, j, fb, SPLA // LANES, CHUNK // LANES, SPLA)
            pltpu.async_copy(fb, acc.at[rowvb.at[j]], semsb, add=True)

        def phase_body(p, carry):
            pltpu.sync_copy(col_hbm.at[wid, p], colv)
            pltpu.sync_copy(rowa_hbm.at[wid, p], rowva)
            pltpu.sync_copy(rowb_hbm.at[wid, p], rowvb)
            pltpu.sync_copy(a_hbm.at[wid, p], av)

            gather(0, bf0, semg0)
            gather(1, bf1, semg1)

            def pair_body(jj, c1):
                j0 = 2 * jj
                j1 = j0 + 1
                gather_wait(j0, bf0, semg0)
                process(bf0, j0)

                @pl.when(j0 + 2 < PC)
                def _():
                    gather(j0 + 2, bf0, semg0)

                gather_wait(j1, bf1, semg1)
                process(bf1, j1)

                @pl.when(j1 + 2 < PC)
                def _():
                    gather(j1 + 2, bf1, semg1)

                return c1

            lax.fori_loop(0, PC // 2, pair_body, 0)

            gather_wait(PC - 1, bf0, semg0)
            process(bf0, PC - 1)
            pltpu.make_async_copy(fa, acc.at[rowva.at[PC - 1]], semsa).wait()
            pltpu.make_async_copy(fb, acc.at[rowvb.at[PC - 1]], semsb).wait()
            return carry

        lax.fori_loop(0, PH, phase_body, 0)

        plsc.subcore_barrier()

        @pl.when(sid < WBT)
        def _():
            pltpu.sync_copy(
                acc.at[pl.ds(sid * WB, WB)],
                out_hbm.at[cid, pl.ds(sid * WB, WB)],
            )

    return agg(v_pk, col2d, rowa, rowb, a2d, zeros)


def _mlp_body(p_ref, v_ref, eps_ref, w1_ref, b1_ref, g1_ref, be1_ref,
              w2_ref, b2_ref, g2_ref, be2_ref, o_ref):
    acc = p_ref[0] + p_ref[1] + eps_ref[...] * v_ref[...]
    h = lax.dot_general(acc, w1_ref[...], (((1,), (1,)), ((), ())),
                        preferred_element_type=jnp.float32) + b1_ref[...]
    m1 = jnp.mean(h, axis=0, keepdims=True)
    var1 = jnp.mean((h - m1) ** 2, axis=0, keepdims=True)
    h = (h - m1) * lax.rsqrt(var1 + 1e-5) * g1_ref[...] + be1_ref[...]
    h = jnp.maximum(h, 0.0)
    o = lax.dot_general(h, w2_ref[...], (((1,), (1,)), ((), ())),
                        preferred_element_type=jnp.float32) + b2_ref[...]
    m2 = jnp.mean(o, axis=0, keepdims=True)
    var2 = jnp.mean((o - m2) ** 2, axis=0, keepdims=True)
    o = (o - m2) * lax.rsqrt(var2 + 1e-5) * g2_ref[...] + be2_ref[...]
    o_ref[...] = jnp.maximum(o, 0.0)


def _mlp(partial, v, epsilon, W1, b1, g1, be1, W2, b2, g2, be2):
    return pl.pallas_call(
        _mlp_body,
        out_shape=jax.ShapeDtypeStruct((N, D), jnp.float32),
    )(partial, v, epsilon,
      W1, b1.reshape(1, -1), g1.reshape(1, -1), be1.reshape(1, -1),
      W2, b2.reshape(1, -1), g2.reshape(1, -1), be2.reshape(1, -1))


_PERM = np.stack([np.arange(DW), np.arange(DW) + DW], axis=1).reshape(-1)


def kernel(v, edge_index, a_values, epsilon, W1, b1, g1, be1, W2, b2, g2, be2):
    row4d = edge_index[0].reshape(NW, PH, PC, CHUNK)
    rowa = row4d[..., :SPLA]
    rowb = row4d[..., SPLA:]
    col2d = edge_index[1].reshape(NW, PH, PC, CHUNK)
    a2d = a_values.reshape(NW, PH, PC, CHUNK)
    v_bf = v.astype(jnp.bfloat16)[:, _PERM]
    v_pk = lax.bitcast_convert_type(v_bf.reshape(N, DW, 2), jnp.int32)
    zeros = jnp.zeros((WB, D), jnp.float32)
    partial = _sc_aggregate(v_pk, col2d, rowa, rowb, a2d, zeros)
    return _mlp(partial, v, epsilon, W1, b1, g1, be1, W2, b2, g2, be2)


# speedup vs baseline: 1.8324x; 1.8324x over previous
import functools

import jax
import jax.numpy as jnp
from jax import lax
from jax.experimental import pallas as pl
from jax.experimental.pallas import tpu as pltpu
from jax.experimental.pallas import tpu_sc as plsc

N = 10000
E = 320000
D = 128

NC = 2
NS = 16
NW = NC * NS
LANES = 16

CHUNK = 80
CPW = E // (NW * CHUNK)
PH = 5
PC = CPW // PH
WB = 1000
WBT = N // WB


def _sc_aggregate(v, col2d, row2d, a2d, zeros):
    mesh = plsc.VectorSubcoreMesh(
        core_axis_name="c", subcore_axis_name="s", num_cores=NC, num_subcores=NS
    )

    @functools.partial(
        pl.kernel,
        out_type=jax.ShapeDtypeStruct((NC, N, D), jnp.float32),
        mesh=mesh,
        scratch_types=[
            pltpu.VMEM((PC, CHUNK), jnp.int32),
            pltpu.VMEM((PC, CHUNK), jnp.int32),
            pltpu.VMEM((PC, CHUNK), jnp.float32),
            pltpu.VMEM((CHUNK, D), jnp.float32),
            pltpu.VMEM((CHUNK, D), jnp.float32),
            pltpu.VMEM_SHARED((N, D), jnp.float32),
            pltpu.SemaphoreType.DMA,
            pltpu.SemaphoreType.DMA,
            pltpu.SemaphoreType.DMA,
            pltpu.SemaphoreType.DMA,
        ],
    )
    def agg(v_hbm, col_hbm, row_hbm, a_hbm, zeros_hbm, out_hbm, colv, rowv, av,
            rows0, rows1, acc, semg0, semg1, sems0, sems1):
        cid = lax.axis_index("c")
        sid = lax.axis_index("s")
        wid = cid * NS + sid

        @pl.when(sid < WBT)
        def _():
            pltpu.sync_copy(zeros_hbm, acc.at[pl.ds(sid * WB, WB)])

        plsc.subcore_barrier()

        def scale(rb, j):
            def group_body(g, c2):
                a_vec = av[j, pl.ds(g * LANES, LANES)]
                for i in range(LANES):
                    s = jnp.full((LANES,), a_vec[i], jnp.float32)
                    e = g * LANES + i
                    for t in range(D // LANES):
                        sl = pl.ds(t * LANES, LANES)
                        rb[e, sl] = rb[e, sl] * s
                return c2

            lax.fori_loop(0, CHUNK // LANES, group_body, 0)

        HC = CHUNK // 2

        def gather(j, rb, sem):
            pltpu.async_copy(
                v_hbm.at[colv.at[j, pl.ds(0, HC)]], rb.at[pl.ds(0, HC)], sem)
            pltpu.async_copy(
                v_hbm.at[colv.at[j, pl.ds(HC, HC)]], rb.at[pl.ds(HC, HC)], sem)

        def gather_wait(j, rb, sem):
            pltpu.make_async_copy(
                v_hbm.at[colv.at[j, pl.ds(0, HC)]], rb.at[pl.ds(0, HC)], sem).wait()
            pltpu.make_async_copy(
                v_hbm.at[colv.at[j, pl.ds(HC, HC)]], rb.at[pl.ds(HC, HC)], sem).wait()

        def scatter(j, rb, sem):
            pltpu.async_copy(rb, acc.at[rowv.at[j]], sem, add=True)

        def scatter_wait(j, rb, sem):
            pltpu.make_async_copy(rb, acc.at[rowv.at[j]], sem).wait()

        def phase_body(p, carry):
            pltpu.sync_copy(col_hbm.at[wid, p], colv)
            pltpu.sync_copy(row_hbm.at[wid, p], rowv)
            pltpu.sync_copy(a_hbm.at[wid, p], av)

            gather(0, rows0, semg0)

            def pair_body(jj, c1):
                j0 = 2 * jj
                j1 = j0 + 1
                j2 = j0 + 2
                gather_wait(j0, rows0, semg0)

                @pl.when(jj > 0)
                def _():
                    scatter_wait(j1 - 2, rows1, sems1)

                gather(j1, rows1, semg1)
                scale(rows0, j0)
                scatter(j0, rows0, sems0)
                gather_wait(j1, rows1, semg1)
                scatter_wait(j0, rows0, sems0)

                @pl.when(j2 < PC)
                def _():
                    gather(j2, rows0, semg0)

                scale(rows1, j1)
                scatter(j1, rows1, sems1)
                return c1

            lax.fori_loop(0, PC // 2, pair_body, 0)

            scatter_wait(PC - 2, rows1, sems1)
            gather_wait(PC - 1, rows0, semg0)
            scale(rows0, PC - 1)
            scatter(PC - 1, rows0, sems0)
            scatter_wait(PC - 1, rows0, sems0)
            return carry

        lax.fori_loop(0, PH, phase_body, 0)

        plsc.subcore_barrier()

        @pl.when(sid < WBT)
        def _():
            pltpu.sync_copy(
                acc.at[pl.ds(sid * WB, WB)],
                out_hbm.at[cid, pl.ds(sid * WB, WB)],
            )

    return agg(v, col2d, row2d, a2d, zeros)


def _mlp_body(p_ref, v_ref, eps_ref, w1_ref, b1_ref, g1_ref, be1_ref,
              w2_ref, b2_ref, g2_ref, be2_ref, o_ref):
    acc = p_ref[0] + p_ref[1] + eps_ref[...] * v_ref[...]
    h = lax.dot_general(acc, w1_ref[...], (((1,), (1,)), ((), ())),
                        preferred_element_type=jnp.float32) + b1_ref[...]
    m1 = jnp.mean(h, axis=0, keepdims=True)
    var1 = jnp.mean((h - m1) ** 2, axis=0, keepdims=True)
    h = (h - m1) * lax.rsqrt(var1 + 1e-5) * g1_ref[...] + be1_ref[...]
    h = jnp.maximum(h, 0.0)
    o = lax.dot_general(h, w2_ref[...], (((1,), (1,)), ((), ())),
                        preferred_element_type=jnp.float32) + b2_ref[...]
    m2 = jnp.mean(o, axis=0, keepdims=True)
    var2 = jnp.mean((o - m2) ** 2, axis=0, keepdims=True)
    o = (o - m2) * lax.rsqrt(var2 + 1e-5) * g2_ref[...] + be2_ref[...]
    o_ref[...] = jnp.maximum(o, 0.0)


def _mlp(partial, v, epsilon, W1, b1, g1, be1, W2, b2, g2, be2):
    return pl.pallas_call(
        _mlp_body,
        out_shape=jax.ShapeDtypeStruct((N, D), jnp.float32),
    )(partial, v, epsilon,
      W1, b1.reshape(1, -1), g1.reshape(1, -1), be1.reshape(1, -1),
      W2, b2.reshape(1, -1), g2.reshape(1, -1), be2.reshape(1, -1))


def kernel(v, edge_index, a_values, epsilon, W1, b1, g1, be1, W2, b2, g2, be2):
    row2d = edge_index[0].reshape(NW, PH, PC, CHUNK)
    col2d = edge_index[1].reshape(NW, PH, PC, CHUNK)
    a2d = a_values.reshape(NW, PH, PC, CHUNK)
    zeros = jnp.zeros((WB, D), jnp.float32)
    partial = _sc_aggregate(v, col2d, row2d, a2d, zeros)
    return _mlp(partial, v, epsilon, W1, b1, g1, be1, W2, b2, g2, be2)


# speedup vs baseline: 1.8480x; 1.0085x over previous
import functools

import jax
import jax.numpy as jnp
from jax import lax
from jax.experimental import pallas as pl
from jax.experimental.pallas import tpu as pltpu
from jax.experimental.pallas import tpu_sc as plsc

N = 10000
E = 320000
D = 128

NC = 2
NS = 16
NW = NC * NS
LANES = 16

CHUNK = 80
CPW = E // (NW * CHUNK)
PH = 5
PC = CPW // PH
WB = 1000
WBT = N // WB


def _sc_aggregate(v, col2d, row2d, a2d):
    mesh = plsc.VectorSubcoreMesh(
        core_axis_name="c", subcore_axis_name="s", num_cores=NC, num_subcores=NS
    )

    @functools.partial(
        pl.kernel,
        out_type=jax.ShapeDtypeStruct((NC, N, D), jnp.float32),
        mesh=mesh,
        scratch_types=[
            pltpu.VMEM((PC, CHUNK), jnp.int32),
            pltpu.VMEM((PC, CHUNK), jnp.int32),
            pltpu.VMEM((PC, CHUNK), jnp.float32),
            pltpu.VMEM((CHUNK, D), jnp.float32),
            pltpu.VMEM((CHUNK, D), jnp.float32),
            pltpu.VMEM((40, D), jnp.float32),
            pltpu.VMEM_SHARED((N, D), jnp.float32),
            pltpu.SemaphoreType.DMA,
            pltpu.SemaphoreType.DMA,
            pltpu.SemaphoreType.DMA,
            pltpu.SemaphoreType.DMA,
        ],
    )
    def agg(v_hbm, col_hbm, row_hbm, a_hbm, out_hbm, colv, rowv, av,
            rows0, rows1, zbuf, acc, semg0, semg1, sems0, sems1):
        cid = lax.axis_index("c")
        sid = lax.axis_index("s")
        wid = cid * NS + sid

        zero16 = jnp.zeros((LANES,), jnp.float32)

        def zero_row(i, carry):
            for t in range(D // LANES):
                zbuf[i, pl.ds(t * LANES, LANES)] = zero16
            return carry

        lax.fori_loop(0, 40, zero_row, 0)

        @pl.when(sid < WBT)
        def _():
            def zero_acc(k, carry):
                pltpu.sync_copy(zbuf, acc.at[pl.ds(sid * WB + k * 40, 40)])
                return carry

            lax.fori_loop(0, WB // 40, zero_acc, 0)

        plsc.subcore_barrier()

        def scale(rb, j):
            def group_body(g, c2):
                a_vec = av[j, pl.ds(g * LANES, LANES)]
                for i in range(LANES):
                    s = jnp.full((LANES,), a_vec[i], jnp.float32)
                    e = g * LANES + i
                    for t in range(D // LANES):
                        sl = pl.ds(t * LANES, LANES)
                        rb[e, sl] = rb[e, sl] * s
                return c2

            lax.fori_loop(0, CHUNK // LANES, group_body, 0)

        HC = CHUNK // 2

        def gather(j, rb, sem):
            pltpu.async_copy(
                v_hbm.at[colv.at[j, pl.ds(0, HC)]], rb.at[pl.ds(0, HC)], sem)
            pltpu.async_copy(
                v_hbm.at[colv.at[j, pl.ds(HC, HC)]], rb.at[pl.ds(HC, HC)], sem)

        def gather_wait(j, rb, sem):
            pltpu.make_async_copy(
                v_hbm.at[colv.at[j, pl.ds(0, HC)]], rb.at[pl.ds(0, HC)], sem).wait()
            pltpu.make_async_copy(
                v_hbm.at[colv.at[j, pl.ds(HC, HC)]], rb.at[pl.ds(HC, HC)], sem).wait()

        def scatter(j, rb, sem):
            pltpu.async_copy(rb, acc.at[rowv.at[j]], sem, add=True)

        def scatter_wait(j, rb, sem):
            pltpu.make_async_copy(rb, acc.at[rowv.at[j]], sem).wait()

        def phase_body(p, carry):
            pltpu.sync_copy(col_hbm.at[wid, p], colv)
            pltpu.sync_copy(row_hbm.at[wid, p], rowv)
            pltpu.sync_copy(a_hbm.at[wid, p], av)

            gather(0, rows0, semg0)

            def pair_body(jj, c1):
                j0 = 2 * jj
                j1 = j0 + 1
                j2 = j0 + 2
                gather_wait(j0, rows0, semg0)

                @pl.when(jj > 0)
                def _():
                    scatter_wait(j1 - 2, rows1, sems1)

                gather(j1, rows1, semg1)
                scale(rows0, j0)
                scatter(j0, rows0, sems0)
                gather_wait(j1, rows1, semg1)
                scatter_wait(j0, rows0, sems0)

                @pl.when(j2 < PC)
                def _():
                    gather(j2, rows0, semg0)

                scale(rows1, j1)
                scatter(j1, rows1, sems1)
                return c1

            lax.fori_loop(0, PC // 2, pair_body, 0)

            scatter_wait(PC - 2, rows1, sems1)
            gather_wait(PC - 1, rows0, semg0)
            scale(rows0, PC - 1)
            scatter(PC - 1, rows0, sems0)
            scatter_wait(PC - 1, rows0, sems0)
            return carry

        lax.fori_loop(0, PH, phase_body, 0)

        plsc.subcore_barrier()

        @pl.when(sid < WBT)
        def _():
            pltpu.sync_copy(
                acc.at[pl.ds(sid * WB, WB)],
                out_hbm.at[cid, pl.ds(sid * WB, WB)],
            )

    return agg(v, col2d, row2d, a2d)


def _mlp_body(p_ref, v_ref, eps_ref, w1_ref, b1_ref, g1_ref, be1_ref,
              w2_ref, b2_ref, g2_ref, be2_ref, o_ref):
    acc = p_ref[0] + p_ref[1] + eps_ref[...] * v_ref[...]
    h = lax.dot_general(acc, w1_ref[...], (((1,), (1,)), ((), ())),
                        preferred_element_type=jnp.float32) + b1_ref[...]
    m1 = jnp.mean(h, axis=0, keepdims=True)
    var1 = jnp.mean((h - m1) ** 2, axis=0, keepdims=True)
    h = (h - m1) * lax.rsqrt(var1 + 1e-5) * g1_ref[...] + be1_ref[...]
    h = jnp.maximum(h, 0.0)
    o = lax.dot_general(h, w2_ref[...], (((1,), (1,)), ((), ())),
                        preferred_element_type=jnp.float32) + b2_ref[...]
    m2 = jnp.mean(o, axis=0, keepdims=True)
    var2 = jnp.mean((o - m2) ** 2, axis=0, keepdims=True)
    o = (o - m2) * lax.rsqrt(var2 + 1e-5) * g2_ref[...] + be2_ref[...]
    o_ref[...] = jnp.maximum(o, 0.0)


def _mlp(partial, v, epsilon, W1, b1, g1, be1, W2, b2, g2, be2):
    return pl.pallas_call(
        _mlp_body,
        out_shape=jax.ShapeDtypeStruct((N, D), jnp.float32),
    )(partial, v, epsilon,
      W1, b1.reshape(1, -1), g1.reshape(1, -1), be1.reshape(1, -1),
      W2, b2.reshape(1, -1), g2.reshape(1, -1), be2.reshape(1, -1))


def kernel(v, edge_index, a_values, epsilon, W1, b1, g1, be1, W2, b2, g2, be2):
    row2d = edge_index[0].reshape(NW, PH, PC, CHUNK)
    col2d = edge_index[1].reshape(NW, PH, PC, CHUNK)
    a2d = a_values.reshape(NW, PH, PC, CHUNK)
    partial = _sc_aggregate(v, col2d, row2d, a2d)
    return _mlp(partial, v, epsilon, W1, b1, g1, be1, W2, b2, g2, be2)
